# baseline (speedup 1.0000x reference)
import functools

import jax
import jax.numpy as jnp
from jax import lax
from jax.experimental import pallas as pl
from jax.experimental.pallas import tpu as pltpu
from jax.experimental.pallas import tpu_sc as plsc

_DIM = 2048
_N_EXPERTS = 16
_TOKENS = 16384
_BLOCK_T = 2048

_NW = 32
_TPW = _TOKENS // _NW
_CHUNKS_PER_BLOCK = _BLOCK_T // _TPW


def _matmul_block(x_ref, w_ref, b_ref, s_out_ref):
    st = jax.lax.dot_general(
        w_ref[...], x_ref[...],
        dimension_numbers=(((1,), (1,)), ((), ())),
        preferred_element_type=jnp.float32,
    ) + b_ref[...]
    for w in range(_CHUNKS_PER_BLOCK):
        s_out_ref[w] = st[:, w * _TPW:(w + 1) * _TPW]


def _tc_scores(x, W, b2):
    grid = (_TOKENS // _BLOCK_T,)
    return pl.pallas_call(
        _matmul_block,
        grid=grid,
        in_specs=[
            pl.BlockSpec((_BLOCK_T, _DIM), lambda i: (i, 0)),
            pl.BlockSpec((_N_EXPERTS, _DIM), lambda i: (0, 0)),
            pl.BlockSpec((_N_EXPERTS, 1), lambda i: (0, 0)),
        ],
        out_specs=pl.BlockSpec(
            (_CHUNKS_PER_BLOCK, _N_EXPERTS, _TPW), lambda i: (i, 0, 0)),
        out_shape=jax.ShapeDtypeStruct((_NW, _N_EXPERTS, _TPW), jnp.float32),
        compiler_params=pltpu.CompilerParams(
            dimension_semantics=("arbitrary",),
        ),
    )(x, W, b2)


def _sc_route_body(s_hbm, w_out_hbm, i_out_hbm, slab_v, wv, iv):
    wid = lax.axis_index("s") * 2 + lax.axis_index("c")
    pltpu.sync_copy(s_hbm.at[wid], slab_v)

    def group(g, _):
        cols = pl.ds(g * 16, 16)
        lane = lax.iota(jnp.int32, 16)
        vs = [slab_v[e, cols] for e in range(_N_EXPERTS)]
        m = vs[0]
        for e in range(1, _N_EXPERTS):
            m = jnp.maximum(m, vs[e])
        es = [jnp.exp(v - m) for v in vs]
        z = es[0]
        for e in range(1, _N_EXPERTS):
            z = z + es[e]
        ps = [ev / z for ev in es]
        v1 = ps[0]
        for e in range(1, _N_EXPERTS):
            v1 = jnp.maximum(v1, ps[e])
        i1 = jnp.where(ps[0] == v1, 0, _N_EXPERTS)
        for e in range(1, _N_EXPERTS):
            i1 = jnp.minimum(i1, jnp.where(ps[e] == v1, e, _N_EXPERTS))
        p2s = [jnp.where(i1 == e, -1.0, ps[e]) for e in range(_N_EXPERTS)]
        v2 = p2s[0]
        for e in range(1, _N_EXPERTS):
            v2 = jnp.maximum(v2, p2s[e])
        i2 = jnp.where(p2s[0] == v2, 0, _N_EXPERTS)
        for e in range(1, _N_EXPERTS):
            i2 = jnp.minimum(i2, jnp.where(p2s[e] == v2, e, _N_EXPERTS))
        s = v1 + v2
        idx1 = lane * 2 + g * 32
        idx2 = idx1 + 1
        plsc.store_scatter(wv, [idx1], v1 / s)
        plsc.store_scatter(wv, [idx2], v2 / s)
        plsc.store_scatter(iv, [idx1], i1)
        plsc.store_scatter(iv, [idx2], i2)
        return _

    lax.fori_loop(0, _TPW // 16, group, 0)
    pltpu.sync_copy(wv, w_out_hbm.at[wid])
    pltpu.sync_copy(iv, i_out_hbm.at[wid])


def _sc_route(scores3):
    mesh = plsc.VectorSubcoreMesh(core_axis_name="c", subcore_axis_name="s")
    fn = functools.partial(
        pl.kernel,
        out_type=[
            jax.ShapeDtypeStruct((_NW, 2 * _TPW), jnp.float32),
            jax.ShapeDtypeStruct((_NW, 2 * _TPW), jnp.int32),
        ],
        mesh=mesh,
        scratch_types=[
            pltpu.VMEM((_N_EXPERTS, _TPW), jnp.float32),
            pltpu.VMEM((2 * _TPW,), jnp.float32),
            pltpu.VMEM((2 * _TPW,), jnp.int32),
        ],
        compiler_params=pltpu.CompilerParams(needs_layout_passes=False),
    )(_sc_route_body)
    return fn(scores3)


def kernel(x, W, b):
    b2 = b.reshape(_N_EXPERTS, 1)
    scores3 = _tc_scores(x, W, b2)
    w3, i3 = _sc_route(scores3)
    weights = w3.reshape(_TOKENS, 2)
    indices = i3.reshape(_TOKENS, 2)
    return (weights, indices)


# speedup vs baseline: 1.6028x; 1.6028x over previous
import jax
import jax.numpy as jnp
from jax.experimental import pallas as pl
from jax.experimental.pallas import tpu as pltpu

_DIM = 2048
_N_EXPERTS = 16
_TOKENS = 16384
_BLOCK_T = 2048


def _gate_block(x_ref, w_ref, b_ref, w_out_ref, i_out_ref):
    st = jax.lax.dot_general(
        w_ref[...], x_ref[...],
        dimension_numbers=(((1,), (1,)), ((), ())),
        preferred_element_type=jnp.float32,
    ) + b_ref[...]
    iota = jax.lax.broadcasted_iota(jnp.int32, st.shape, 0).astype(jnp.float32)
    m = jnp.max(st, axis=0, keepdims=True)
    e = jnp.exp(st - m)
    p = e / jnp.sum(e, axis=0, keepdims=True)
    v1 = jnp.max(p, axis=0, keepdims=True)
    i1 = jnp.min(jnp.where(p == v1, iota, float(_N_EXPERTS)),
                 axis=0, keepdims=True)
    p2 = jnp.where(iota == i1, -1.0, p)
    v2 = jnp.max(p2, axis=0, keepdims=True)
    i2 = jnp.min(jnp.where(p2 == v2, iota, float(_N_EXPERTS)),
                 axis=0, keepdims=True)
    s = v1 + v2
    w2t = jnp.concatenate([v1 / s, v2 / s], axis=0)
    i2t = jnp.concatenate([i1, i2], axis=0).astype(jnp.int32)
    w_out_ref[...] = jnp.transpose(w2t)
    i_out_ref[...] = jnp.transpose(i2t)


def kernel(x, W, b):
    b2 = b.reshape(_N_EXPERTS, 1)
    grid = (_TOKENS // _BLOCK_T,)
    weights, indices = pl.pallas_call(
        _gate_block,
        grid=grid,
        in_specs=[
            pl.BlockSpec((_BLOCK_T, _DIM), lambda i: (i, 0)),
            pl.BlockSpec((_N_EXPERTS, _DIM), lambda i: (0, 0)),
            pl.BlockSpec((_N_EXPERTS, 1), lambda i: (0, 0)),
        ],
        out_specs=[
            pl.BlockSpec((_BLOCK_T, 2), lambda i: (i, 0)),
            pl.BlockSpec((_BLOCK_T, 2), lambda i: (i, 0)),
        ],
        out_shape=[
            jax.ShapeDtypeStruct((_TOKENS, 2), jnp.float32),
            jax.ShapeDtypeStruct((_TOKENS, 2), jnp.int32),
        ],
        compiler_params=pltpu.CompilerParams(
            dimension_semantics=("parallel",),
        ),
    )(x, W, b2)
    return (weights, indices)


# speedup vs baseline: 2.0547x; 1.2819x over previous
import jax
import jax.numpy as jnp
from jax.experimental import pallas as pl
from jax.experimental.pallas import tpu as pltpu

_DIM = 2048
_N_EXPERTS = 16
_TOKENS = 16384
_CHUNK = 2048
_NBUF = 3
_NCHUNKS = _TOKENS // _CHUNK


def _tail(st, w_out_ref, i_out_ref, base):
    iota = jax.lax.broadcasted_iota(jnp.int32, st.shape, 0).astype(jnp.float32)
    m = jnp.max(st, axis=0, keepdims=True)
    e = jnp.exp(st - m)
    p = e / jnp.sum(e, axis=0, keepdims=True)
    v1 = jnp.max(p, axis=0, keepdims=True)
    i1 = jnp.min(jnp.where(p == v1, iota, float(_N_EXPERTS)),
                 axis=0, keepdims=True)
    p2 = jnp.where(iota == i1, -1.0, p)
    v2 = jnp.max(p2, axis=0, keepdims=True)
    i2 = jnp.min(jnp.where(p2 == v2, iota, float(_N_EXPERTS)),
                 axis=0, keepdims=True)
    s = v1 + v2
    w2t = jnp.concatenate([v1 / s, v2 / s], axis=0)
    i2t = jnp.concatenate([i1, i2], axis=0).astype(jnp.int32)
    w_out_ref[:, pl.ds(base, _CHUNK)] = w2t
    i_out_ref[:, pl.ds(base, _CHUNK)] = i2t


def _body(x_hbm, w_ref, b_ref, w_out_ref, i_out_ref, ring, sems):
    def copy(c):
        return pltpu.make_async_copy(
            x_hbm.at[pl.ds(c * _CHUNK, _CHUNK), :],
            ring.at[c % _NBUF], sems.at[c % _NBUF])

    for c in range(_NBUF):
        copy(c).start()
    for c in range(_NCHUNKS):
        copy(c).wait()
        st = jax.lax.dot_general(
            w_ref[...], ring[c % _NBUF],
            dimension_numbers=(((1,), (1,)), ((), ())),
            preferred_element_type=jnp.float32,
        ) + b_ref[...]
        if c + _NBUF < _NCHUNKS:
            copy(c + _NBUF).start()
        _tail(st, w_out_ref, i_out_ref, c * _CHUNK)


def kernel(x, W, b):
    b2 = b.reshape(_N_EXPERTS, 1)
    weights, indices = pl.pallas_call(
        _body,
        in_specs=[
            pl.BlockSpec(memory_space=pltpu.HBM),
            pl.BlockSpec(memory_space=pltpu.VMEM),
            pl.BlockSpec(memory_space=pltpu.VMEM),
        ],
        out_specs=[
            pl.BlockSpec(memory_space=pltpu.VMEM),
            pl.BlockSpec(memory_space=pltpu.VMEM),
        ],
        out_shape=[
            jax.ShapeDtypeStruct((2, _TOKENS), jnp.float32),
            jax.ShapeDtypeStruct((2, _TOKENS), jnp.int32),
        ],
        scratch_shapes=[
            pltpu.VMEM((_NBUF, _CHUNK, _DIM), jnp.float32),
            pltpu.SemaphoreType.DMA((_NBUF,)),
        ],
    )(x, W, b2)
    return (jnp.transpose(weights), jnp.transpose(indices))


# speedup vs baseline: 2.0605x; 1.0028x over previous
import jax
import jax.numpy as jnp
from jax.experimental import pallas as pl
from jax.experimental.pallas import tpu as pltpu

_DIM = 2048
_N_EXPERTS = 16
_TOKENS = 16384
_CHUNK = 1024
_NBUF = 6
_NCHUNKS = _TOKENS // _CHUNK


def _tail(st, w_out_ref, i_out_ref, base):
    iota = jax.lax.broadcasted_iota(jnp.int32, st.shape, 0).astype(jnp.float32)
    m = jnp.max(st, axis=0, keepdims=True)
    e = jnp.exp(st - m)
    p = e / jnp.sum(e, axis=0, keepdims=True)
    v1 = jnp.max(p, axis=0, keepdims=True)
    i1 = jnp.min(jnp.where(p == v1, iota, float(_N_EXPERTS)),
                 axis=0, keepdims=True)
    p2 = jnp.where(iota == i1, -1.0, p)
    v2 = jnp.max(p2, axis=0, keepdims=True)
    i2 = jnp.min(jnp.where(p2 == v2, iota, float(_N_EXPERTS)),
                 axis=0, keepdims=True)
    s = v1 + v2
    w2t = jnp.concatenate([v1 / s, v2 / s], axis=0)
    i2t = jnp.concatenate([i1, i2], axis=0).astype(jnp.int32)
    w_out_ref[:, pl.ds(base, _CHUNK)] = w2t
    i_out_ref[:, pl.ds(base, _CHUNK)] = i2t


def _body(x_hbm, w_ref, b_ref, w_out_ref, i_out_ref, ring, sems):
    def copy(c):
        return pltpu.make_async_copy(
            x_hbm.at[pl.ds(c * _CHUNK, _CHUNK), :],
            ring.at[c % _NBUF], sems.at[c % _NBUF])

    for c in range(_NBUF):
        copy(c).start()
    for c in range(_NCHUNKS):
        copy(c).wait()
        st = jax.lax.dot_general(
            w_ref[...], ring[c % _NBUF],
            dimension_numbers=(((1,), (1,)), ((), ())),
            preferred_element_type=jnp.float32,
        ) + b_ref[...]
        if c + _NBUF < _NCHUNKS:
            copy(c + _NBUF).start()
        _tail(st, w_out_ref, i_out_ref, c * _CHUNK)


def kernel(x, W, b):
    b2 = b.reshape(_N_EXPERTS, 1)
    weights, indices = pl.pallas_call(
        _body,
        in_specs=[
            pl.BlockSpec(memory_space=pltpu.HBM),
            pl.BlockSpec(memory_space=pltpu.VMEM),
            pl.BlockSpec(memory_space=pltpu.VMEM),
        ],
        out_specs=[
            pl.BlockSpec(memory_space=pltpu.VMEM),
            pl.BlockSpec(memory_space=pltpu.VMEM),
        ],
        out_shape=[
            jax.ShapeDtypeStruct((2, _TOKENS), jnp.float32),
            jax.ShapeDtypeStruct((2, _TOKENS), jnp.int32),
        ],
        scratch_shapes=[
            pltpu.VMEM((_NBUF, _CHUNK, _DIM), jnp.float32),
            pltpu.SemaphoreType.DMA((_NBUF,)),
        ],
    )(x, W, b2)
    return (jnp.transpose(weights), jnp.transpose(indices))
